# baseline (speedup 1.0000x reference)
import functools

import jax
import jax.numpy as jnp
from jax import lax
from jax.experimental import pallas as pl
from jax.experimental.pallas import tpu as pltpu
from jax.experimental.pallas import tpu_sc as plsc

B = 16384
EMB = 32
OUT_D = EMB + 1
VOCAB = 1000000
PCOLS = 128
PROWS = PCOLS // 4
NBLK = -(-VOCAB // PCOLS)
VP = NBLK * PROWS
NC, NS, L = 2, 16, 16
NW = NC * NS
BW = B // NW
CHUNK = 128
NCH = BW // CHUNK
NSLOT = 2

_mesh = plsc.VectorSubcoreMesh(core_axis_name="c", subcore_axis_name="s")


def _pack_body(tT_ref, out_ref):
  x = tT_ref[...]
  cols = jnp.broadcast_to(
      4 * jnp.arange(PROWS, dtype=jnp.int32)[None, :], (EMB, PROWS))
  out_ref[...] = jnp.concatenate(
      [jnp.take_along_axis(x, cols + q, axis=1).T for q in range(4)], axis=1)


def _pack_call(tT):
  return pl.pallas_call(
      _pack_body,
      grid=(NBLK,),
      in_specs=[pl.BlockSpec((EMB, PCOLS), lambda i: (0, i))],
      out_specs=pl.BlockSpec((PROWS, 128), lambda i: (i, 0)),
      out_shape=jax.ShapeDtypeStruct((VP, 128), jnp.float32),
  )(tT)


def _prep_rowidx(idx_ref, rowidx_ref):
  lanes = lax.iota(jnp.int32, L)

  def group(g, _):
    sv = g * L + lanes
    ids = plsc.load_gather(idx_ref, [sv])
    plsc.store_scatter(rowidx_ref, [sv], jax.lax.shift_right_logical(ids, 2))
    return 0

  lax.fori_loop(0, BW // L, group, 0, unroll=False)


def _extract_chunk(idx_ref, rows_ref, feat_ref, j):
  lanes = lax.iota(jnp.int32, L)
  zeros = jnp.zeros((L,), jnp.int32)

  def group(g, _):
    sv = j * CHUNK + g * L + lanes
    lid = g * L + lanes
    ids = plsc.load_gather(idx_ref, [sv])
    plsc.store_scatter(feat_ref, [zeros, sv], ids.astype(jnp.float32))
    colbase = (ids & 3) * EMB
    for r in range(EMB):
      vals = plsc.load_gather(rows_ref, [lid, colbase + r])
      plsc.store_scatter(feat_ref, [zeros + (1 + r), sv], vals)
    return 0

  lax.fori_loop(0, CHUNK // L, group, 0, unroll=False)


def _sc_body(uids, iids, ut4, it4, uoutT, ioutT,
             uidx, iidx, urowidx, irowidx, urows, irows, ufeat, ifeat,
             usems, isems):
  wid = lax.axis_index("s") * NC + lax.axis_index("c")
  base = wid * BW

  pltpu.sync_copy(uids.at[pl.ds(base, BW)], uidx)
  pltpu.sync_copy(iids.at[pl.ds(base, BW)], iidx)
  _prep_rowidx(uidx, urowidx)
  _prep_rowidx(iidx, irowidx)

  def fire(tbl, rowidx, rows, sems, j):
    return pltpu.async_copy(
        tbl.at[rowidx.at[pl.ds(j * CHUNK, CHUNK)]],
        rows.at[j % NSLOT], sems.at[j % NSLOT])

  ucopies = [fire(ut4, urowidx, urows, usems, j) for j in range(NSLOT)]
  icopies = [fire(it4, irowidx, irows, isems, j) for j in range(NSLOT)]

  for j in range(NCH):
    ucopies[j].wait()
    _extract_chunk(uidx, urows.at[j % NSLOT], ufeat, j)
    if j + NSLOT < NCH:
      ucopies.append(fire(ut4, urowidx, urows, usems, j + NSLOT))
  pltpu.sync_copy(ufeat, uoutT.at[:, pl.ds(base, BW)])

  for j in range(NCH):
    icopies[j].wait()
    _extract_chunk(iidx, irows.at[j % NSLOT], ifeat, j)
    if j + NSLOT < NCH:
      icopies.append(fire(it4, irowidx, irows, isems, j + NSLOT))
  pltpu.sync_copy(ifeat, ioutT.at[:, pl.ds(base, BW)])


_sc_call = functools.partial(
    pl.kernel,
    out_type=[
        jax.ShapeDtypeStruct((OUT_D, B), jnp.float32),
        jax.ShapeDtypeStruct((OUT_D, B), jnp.float32),
    ],
    mesh=_mesh,
    scratch_types=[
        pltpu.VMEM((BW,), jnp.int32),
        pltpu.VMEM((BW,), jnp.int32),
        pltpu.VMEM((BW,), jnp.int32),
        pltpu.VMEM((BW,), jnp.int32),
        pltpu.VMEM((NSLOT, CHUNK, 128), jnp.float32),
        pltpu.VMEM((NSLOT, CHUNK, 128), jnp.float32),
        pltpu.VMEM((OUT_D, BW), jnp.float32),
        pltpu.VMEM((OUT_D, BW), jnp.float32),
        pltpu.SemaphoreType.DMA((NSLOT,)),
        pltpu.SemaphoreType.DMA((NSLOT,)),
    ],
    compiler_params=pltpu.CompilerParams(needs_layout_passes=False),
)(_sc_body)


@jax.jit
def kernel(user_id, item_id, user_table, item_table):
  uids = user_id.reshape(B).astype(jnp.int32)
  iids = item_id.reshape(B).astype(jnp.int32)
  ut4 = _pack_call(user_table.T)
  it4 = _pack_call(item_table.T)
  uT, iT = _sc_call(uids, iids, ut4, it4)
  return uT.T, iT.T


# speedup vs baseline: 3.2649x; 3.2649x over previous
import functools

import jax
import jax.numpy as jnp
from jax import lax
from jax.experimental import pallas as pl
from jax.experimental.pallas import tpu as pltpu
from jax.experimental.pallas import tpu_sc as plsc

B = 16384
EMB = 32
OUT_D = EMB + 1
VOCAB = 1000000
PCOLS = 2048
PROWS = PCOLS // 4
NBLK = -(-VOCAB // PCOLS)
VP = NBLK * PROWS
NC, NS, L = 2, 16, 16
NW = NC * NS
BW = B // NW
CHUNK = 128
NCH = BW // CHUNK
NSLOT = 2

_mesh = plsc.VectorSubcoreMesh(core_axis_name="c", subcore_axis_name="s")


def _pack_body(tT_ref, out_ref):
  x = tT_ref[...]
  cols = jnp.broadcast_to(
      4 * jnp.arange(32, dtype=jnp.int32)[None, :], (EMB, 32))
  outs = []
  for k in range(PCOLS // 128):
    xk = x[:, k * 128:(k + 1) * 128]
    outs.append(jnp.concatenate(
        [jnp.take_along_axis(xk, cols + q, axis=1).T for q in range(4)],
        axis=1))
  out_ref[...] = jnp.concatenate(outs, axis=0)


def _pack_call(tT):
  return pl.pallas_call(
      _pack_body,
      grid=(NBLK,),
      in_specs=[pl.BlockSpec((EMB, PCOLS), lambda i: (0, i))],
      out_specs=pl.BlockSpec((PROWS, 128), lambda i: (i, 0)),
      out_shape=jax.ShapeDtypeStruct((VP, 128), jnp.float32),
  )(tT)


def _prep_rowidx(idx_ref, rowidx_ref):
  lanes = lax.iota(jnp.int32, L)

  def group(g, _):
    sv = g * L + lanes
    ids = plsc.load_gather(idx_ref, [sv])
    plsc.store_scatter(rowidx_ref, [sv], jax.lax.shift_right_logical(ids, 2))
    return 0

  lax.fori_loop(0, BW // L, group, 0, unroll=False)


def _extract_chunk(idx_ref, rows_ref, feat_ref, j):
  lanes = lax.iota(jnp.int32, L)
  zeros = jnp.zeros((L,), jnp.int32)

  def group(g, _):
    sv = j * CHUNK + g * L + lanes
    lid = g * L + lanes
    ids = plsc.load_gather(idx_ref, [sv])
    plsc.store_scatter(feat_ref, [zeros, sv], ids.astype(jnp.float32))
    colbase = (ids & 3) * EMB
    for r in range(EMB):
      vals = plsc.load_gather(rows_ref, [lid, colbase + r])
      plsc.store_scatter(feat_ref, [zeros + (1 + r), sv], vals)
    return 0

  lax.fori_loop(0, CHUNK // L, group, 0, unroll=False)


def _sc_body(uids, iids, ut4, it4, uoutT, ioutT,
             uidx, iidx, urowidx, irowidx, urows, irows, ufeat, ifeat,
             usems, isems):
  wid = lax.axis_index("s") * NC + lax.axis_index("c")
  base = wid * BW

  pltpu.sync_copy(uids.at[pl.ds(base, BW)], uidx)
  pltpu.sync_copy(iids.at[pl.ds(base, BW)], iidx)
  _prep_rowidx(uidx, urowidx)
  _prep_rowidx(iidx, irowidx)

  def fire(tbl, rowidx, rows, sems, j):
    return pltpu.async_copy(
        tbl.at[rowidx.at[pl.ds(j * CHUNK, CHUNK)]],
        rows.at[j % NSLOT], sems.at[j % NSLOT])

  ucopies = [fire(ut4, urowidx, urows, usems, j) for j in range(NSLOT)]
  icopies = [fire(it4, irowidx, irows, isems, j) for j in range(NSLOT)]

  for j in range(NCH):
    ucopies[j].wait()
    _extract_chunk(uidx, urows.at[j % NSLOT], ufeat, j)
    if j + NSLOT < NCH:
      ucopies.append(fire(ut4, urowidx, urows, usems, j + NSLOT))
  pltpu.sync_copy(ufeat, uoutT.at[:, pl.ds(base, BW)])

  for j in range(NCH):
    icopies[j].wait()
    _extract_chunk(iidx, irows.at[j % NSLOT], ifeat, j)
    if j + NSLOT < NCH:
      icopies.append(fire(it4, irowidx, irows, isems, j + NSLOT))
  pltpu.sync_copy(ifeat, ioutT.at[:, pl.ds(base, BW)])


_sc_call = functools.partial(
    pl.kernel,
    out_type=[
        jax.ShapeDtypeStruct((OUT_D, B), jnp.float32),
        jax.ShapeDtypeStruct((OUT_D, B), jnp.float32),
    ],
    mesh=_mesh,
    scratch_types=[
        pltpu.VMEM((BW,), jnp.int32),
        pltpu.VMEM((BW,), jnp.int32),
        pltpu.VMEM((BW,), jnp.int32),
        pltpu.VMEM((BW,), jnp.int32),
        pltpu.VMEM((NSLOT, CHUNK, 128), jnp.float32),
        pltpu.VMEM((NSLOT, CHUNK, 128), jnp.float32),
        pltpu.VMEM((OUT_D, BW), jnp.float32),
        pltpu.VMEM((OUT_D, BW), jnp.float32),
        pltpu.SemaphoreType.DMA((NSLOT,)),
        pltpu.SemaphoreType.DMA((NSLOT,)),
    ],
    compiler_params=pltpu.CompilerParams(needs_layout_passes=False),
)(_sc_body)


@jax.jit
def kernel(user_id, item_id, user_table, item_table):
  uids = user_id.reshape(B).astype(jnp.int32)
  iids = item_id.reshape(B).astype(jnp.int32)
  ut4 = _pack_call(user_table.T)
  it4 = _pack_call(item_table.T)
  uT, iT = _sc_call(uids, iids, ut4, it4)
  return uT.T, iT.T


# speedup vs baseline: 3.7465x; 1.1475x over previous
import functools

import jax
import jax.numpy as jnp
from jax import lax
from jax.experimental import pallas as pl
from jax.experimental.pallas import tpu as pltpu
from jax.experimental.pallas import tpu_sc as plsc

B = 16384
EMB = 32
OUT_D = EMB + 1
VOCAB = 1000000
TFULL = VOCAB // 128
VP = (TFULL + 1) * 32
NC, NS, L = 2, 16, 16
NW = NC * NS
BW = B // NW
CHUNK = 128
NCH = BW // CHUNK
NSLOT = 2

_mesh = plsc.VectorSubcoreMesh(core_axis_name="c", subcore_axis_name="s")


def _pack_one(tT, tail4, packed, xbuf, pkbuf, wid, nb):
  lanes = lax.iota(jnp.int32, L)
  zeros = jnp.zeros((L,), jnp.int32)

  def blk(k, _):
    t = wid + k * NW
    c0 = pl.multiple_of(t * 128, 128)
    pltpu.sync_copy(tT.at[:, pl.ds(c0, 128)], xbuf)
    for m in range(32):
      for h in range(8):
        cv = lanes + (h % 2) * 16
        vals = plsc.load_gather(xbuf, [cv, zeros + (4 * m + h // 2)])
        plsc.store_scatter(pkbuf, [zeros + m, lanes + h * 16], vals)
    r0 = pl.multiple_of(t * 32, 32)
    pltpu.sync_copy(pkbuf, packed.at[pl.ds(r0, 32)])
    return 0

  lax.fori_loop(0, nb, blk, 0, unroll=False)

  @pl.when(wid == 0)
  def _tail():
    pltpu.sync_copy(tail4, pkbuf.at[pl.ds(0, 16)])
    pltpu.sync_copy(pkbuf.at[pl.ds(0, 16)],
                    packed.at[pl.ds(TFULL * 32, 16)])


def _pack_body(utT, itT, utail4, itail4, upacked, ipacked, xbuf, pkbuf):
  wid = lax.axis_index("s") * NC + lax.axis_index("c")
  nb = 244 + (wid < TFULL - 244 * NW).astype(jnp.int32)
  _pack_one(utT, utail4, upacked, xbuf, pkbuf, wid, nb)
  _pack_one(itT, itail4, ipacked, xbuf, pkbuf, wid, nb)


_pack_call = functools.partial(
    pl.kernel,
    out_type=[
        jax.ShapeDtypeStruct((VP, 128), jnp.float32),
        jax.ShapeDtypeStruct((VP, 128), jnp.float32),
    ],
    mesh=_mesh,
    scratch_types=[
        pltpu.VMEM((EMB, 128), jnp.float32),
        pltpu.VMEM((32, 128), jnp.float32),
    ],
    compiler_params=pltpu.CompilerParams(needs_layout_passes=False),
)(_pack_body)


def _prep_rowidx(idx_ref, rowidx_ref):
  lanes = lax.iota(jnp.int32, L)

  def group(g, _):
    sv = g * L + lanes
    ids = plsc.load_gather(idx_ref, [sv])
    plsc.store_scatter(rowidx_ref, [sv], jax.lax.shift_right_logical(ids, 2))
    return 0

  lax.fori_loop(0, BW // L, group, 0, unroll=False)


def _extract_chunk(idx_ref, rows_ref, feat_ref, j):
  lanes = lax.iota(jnp.int32, L)
  zeros = jnp.zeros((L,), jnp.int32)

  def group(g, _):
    sv = j * CHUNK + g * L + lanes
    lid = g * L + lanes
    ids = plsc.load_gather(idx_ref, [sv])
    plsc.store_scatter(feat_ref, [zeros, sv], ids.astype(jnp.float32))
    colbase = (ids & 3) * EMB
    for r in range(EMB):
      vals = plsc.load_gather(rows_ref, [lid, colbase + r])
      plsc.store_scatter(feat_ref, [zeros + (1 + r), sv], vals)
    return 0

  lax.fori_loop(0, CHUNK // L, group, 0, unroll=False)


def _sc_body(uids, iids, ut4, it4, uoutT, ioutT,
             uidx, iidx, urowidx, irowidx, urows, irows, ufeat, ifeat,
             usems, isems):
  wid = lax.axis_index("s") * NC + lax.axis_index("c")
  base = wid * BW

  pltpu.sync_copy(uids.at[pl.ds(base, BW)], uidx)
  pltpu.sync_copy(iids.at[pl.ds(base, BW)], iidx)
  _prep_rowidx(uidx, urowidx)
  _prep_rowidx(iidx, irowidx)

  def fire(tbl, rowidx, rows, sems, j):
    return pltpu.async_copy(
        tbl.at[rowidx.at[pl.ds(j * CHUNK, CHUNK)]],
        rows.at[j % NSLOT], sems.at[j % NSLOT])

  ucopies = [fire(ut4, urowidx, urows, usems, j) for j in range(NSLOT)]
  icopies = [fire(it4, irowidx, irows, isems, j) for j in range(NSLOT)]

  for j in range(NCH):
    ucopies[j].wait()
    _extract_chunk(uidx, urows.at[j % NSLOT], ufeat, j)
    if j + NSLOT < NCH:
      ucopies.append(fire(ut4, urowidx, urows, usems, j + NSLOT))
  pltpu.sync_copy(ufeat, uoutT.at[:, pl.ds(base, BW)])

  for j in range(NCH):
    icopies[j].wait()
    _extract_chunk(iidx, irows.at[j % NSLOT], ifeat, j)
    if j + NSLOT < NCH:
      icopies.append(fire(it4, irowidx, irows, isems, j + NSLOT))
  pltpu.sync_copy(ifeat, ioutT.at[:, pl.ds(base, BW)])


_sc_call = functools.partial(
    pl.kernel,
    out_type=[
        jax.ShapeDtypeStruct((OUT_D, B), jnp.float32),
        jax.ShapeDtypeStruct((OUT_D, B), jnp.float32),
    ],
    mesh=_mesh,
    scratch_types=[
        pltpu.VMEM((BW,), jnp.int32),
        pltpu.VMEM((BW,), jnp.int32),
        pltpu.VMEM((BW,), jnp.int32),
        pltpu.VMEM((BW,), jnp.int32),
        pltpu.VMEM((NSLOT, CHUNK, 128), jnp.float32),
        pltpu.VMEM((NSLOT, CHUNK, 128), jnp.float32),
        pltpu.VMEM((OUT_D, BW), jnp.float32),
        pltpu.VMEM((OUT_D, BW), jnp.float32),
        pltpu.SemaphoreType.DMA((NSLOT,)),
        pltpu.SemaphoreType.DMA((NSLOT,)),
    ],
    compiler_params=pltpu.CompilerParams(needs_layout_passes=False),
)(_sc_body)


@jax.jit
def kernel(user_id, item_id, user_table, item_table):
  uids = user_id.reshape(B).astype(jnp.int32)
  iids = item_id.reshape(B).astype(jnp.int32)
  utail4 = user_table[TFULL * 128:].reshape(16, 128)
  itail4 = item_table[TFULL * 128:].reshape(16, 128)
  ut4, it4 = _pack_call(user_table.T, item_table.T, utail4, itail4)
  uT, iT = _sc_call(uids, iids, ut4, it4)
  return uT.T, iT.T


# speedup vs baseline: 9.5723x; 2.5550x over previous
import functools

import jax
import jax.numpy as jnp
from jax import lax
from jax.experimental import pallas as pl
from jax.experimental.pallas import tpu as pltpu
from jax.experimental.pallas import tpu_sc as plsc

B = 16384
EMB = 32
OUT_D = EMB + 1
VOCAB4 = 250000
NC, NS, L = 2, 16, 16
NW = NC * NS
BW = B // NW
CHUNK = 128
NCH = BW // CHUNK
NSLOT = 2

_mesh = plsc.VectorSubcoreMesh(core_axis_name="c", subcore_axis_name="s")


def _prep_rowidx(idx_ref, rowidx_ref):
  lanes = lax.iota(jnp.int32, L)

  def group(g, _):
    sv = g * L + lanes
    ids = plsc.load_gather(idx_ref, [sv])
    plsc.store_scatter(rowidx_ref, [sv], jax.lax.shift_right_logical(ids, 2))
    return 0

  lax.fori_loop(0, BW // L, group, 0, unroll=False)


def _extract_chunk(idx_ref, rows_ref, feat_ref, j):
  lanes = lax.iota(jnp.int32, L)
  zeros = jnp.zeros((L,), jnp.int32)

  def group(g, _):
    sv = j * CHUNK + g * L + lanes
    lid = g * L + lanes
    ids = plsc.load_gather(idx_ref, [sv])
    plsc.store_scatter(feat_ref, [zeros, sv], ids.astype(jnp.float32))
    colbase = (ids & 3) * EMB
    for r in range(EMB):
      vals = plsc.load_gather(rows_ref, [lid, colbase + r])
      plsc.store_scatter(feat_ref, [zeros + (1 + r), sv], vals)
    return 0

  lax.fori_loop(0, CHUNK // L, group, 0, unroll=False)


def _body(uids, iids, ut4, it4, uoutT, ioutT,
          uidx, iidx, urowidx, irowidx, urows, irows, ufeat, ifeat,
          usems, isems):
  wid = lax.axis_index("s") * NC + lax.axis_index("c")
  base = wid * BW

  pltpu.sync_copy(uids.at[pl.ds(base, BW)], uidx)
  pltpu.sync_copy(iids.at[pl.ds(base, BW)], iidx)
  _prep_rowidx(uidx, urowidx)
  _prep_rowidx(iidx, irowidx)

  def fire(tbl, rowidx, rows, sems, j):
    return pltpu.async_copy(
        tbl.at[rowidx.at[pl.ds(j * CHUNK, CHUNK)]],
        rows.at[j % NSLOT], sems.at[j % NSLOT])

  ucopies = [fire(ut4, urowidx, urows, usems, j) for j in range(NSLOT)]
  icopies = [fire(it4, irowidx, irows, isems, j) for j in range(NSLOT)]

  for j in range(NCH):
    ucopies[j].wait()
    _extract_chunk(uidx, urows.at[j % NSLOT], ufeat, j)
    if j + NSLOT < NCH:
      ucopies.append(fire(ut4, urowidx, urows, usems, j + NSLOT))
  pltpu.sync_copy(ufeat, uoutT.at[:, pl.ds(base, BW)])

  for j in range(NCH):
    icopies[j].wait()
    _extract_chunk(iidx, irows.at[j % NSLOT], ifeat, j)
    if j + NSLOT < NCH:
      icopies.append(fire(it4, irowidx, irows, isems, j + NSLOT))
  pltpu.sync_copy(ifeat, ioutT.at[:, pl.ds(base, BW)])


_sc_call = functools.partial(
    pl.kernel,
    out_type=[
        jax.ShapeDtypeStruct((OUT_D, B), jnp.float32),
        jax.ShapeDtypeStruct((OUT_D, B), jnp.float32),
    ],
    mesh=_mesh,
    scratch_types=[
        pltpu.VMEM((BW,), jnp.int32),
        pltpu.VMEM((BW,), jnp.int32),
        pltpu.VMEM((BW,), jnp.int32),
        pltpu.VMEM((BW,), jnp.int32),
        pltpu.VMEM((NSLOT, CHUNK, 128), jnp.float32),
        pltpu.VMEM((NSLOT, CHUNK, 128), jnp.float32),
        pltpu.VMEM((OUT_D, BW), jnp.float32),
        pltpu.VMEM((OUT_D, BW), jnp.float32),
        pltpu.SemaphoreType.DMA((NSLOT,)),
        pltpu.SemaphoreType.DMA((NSLOT,)),
    ],
    compiler_params=pltpu.CompilerParams(needs_layout_passes=False),
)(_body)


@jax.jit
def kernel(user_id, item_id, user_table, item_table):
  uids = user_id.reshape(B).astype(jnp.int32)
  iids = item_id.reshape(B).astype(jnp.int32)
  ut4 = user_table.reshape(VOCAB4, 128)
  it4 = item_table.reshape(VOCAB4, 128)
  uT, iT = _sc_call(uids, iids, ut4, it4)
  return uT.T, iT.T
